# baseline (speedup 1.0000x reference)
import functools

import jax
import jax.numpy as jnp
from jax import lax
from jax.experimental import pallas as pl
from jax.experimental.pallas import tpu as pltpu
from jax.experimental.pallas import tpu_sc as plsc

N = 10000
E = 320000
D = 128
HID = 64
W = 128

NC, NS, L = 2, 16, 16
NW = NC * NS
EPW = E // NW
B = 100
NB = EPW // B
GPB = 2
NG = NB // GPB
GR = GPB * B

BG = 96
EPAD = 322560
EPWG = EPAD // NW
NGG = EPWG // BG
EPL = EPW // L
NPT = N // NS
NPA = 624
NTAIL = N - NS * NPA

_mesh = plsc.VectorSubcoreMesh(core_axis_name="c", subcore_axis_name="s")


def _wid():
    return lax.axis_index("s") * NC + lax.axis_index("c")


def _prep_body(x_ref, win_ref, bin_ref, g_ref, b_ref, h_ref):
    h = jnp.dot(x_ref[...].astype(jnp.bfloat16), win_ref[...],
                preferred_element_type=jnp.float32)
    h = h + bin_ref[...]
    mean = jnp.mean(h, axis=0, keepdims=True)
    var = jnp.mean((h - mean) ** 2, axis=0, keepdims=True)
    hn = g_ref[...] * (h - mean) * lax.rsqrt(var + 1e-5) + b_ref[...]
    t = jnp.tanh(hn)
    h_ref[...] = jnp.concatenate([t, jnp.zeros_like(t)], axis=1)


def _prep(x, W_in, b_in, gamma, beta):
    return pl.pallas_call(
        _prep_body,
        out_shape=jax.ShapeDtypeStruct((N, W), jnp.float32),
    )(x, W_in, b_in, gamma, beta)


_BE = 1920


def _mlp_body(xi_ref, xj_ref, w1c_ref, w1b_ref, b1_ref, w2_ref, b2_ref, o_ref):
    xi = xi_ref[...]
    xj = xj_ref[...]
    m1 = jnp.dot(xi.astype(jnp.bfloat16), w1c_ref[...],
                 preferred_element_type=jnp.float32)
    m1 = m1 + jnp.dot((xj - xi).astype(jnp.bfloat16), w1b_ref[...],
                      preferred_element_type=jnp.float32)
    m1 = jnp.maximum(m1 + b1_ref[...], 0.0)
    m2 = jnp.dot(m1.astype(jnp.bfloat16), w2_ref[...],
                 preferred_element_type=jnp.float32)
    o_ref[...] = jnp.maximum(m2 + b2_ref[...], 0.0)


def _mlp(xi, xj, W1cp, W1bp, b1, W2p, b2p):
    nblk = EPAD // _BE
    blk = lambda i: (i, 0)
    full = lambda i: (0, 0)
    return pl.pallas_call(
        _mlp_body,
        grid=(nblk,),
        in_specs=[
            pl.BlockSpec((_BE, W), blk),
            pl.BlockSpec((_BE, W), blk),
            pl.BlockSpec((W, W), full),
            pl.BlockSpec((W, W), full),
            pl.BlockSpec((1, W), full),
            pl.BlockSpec((W, W), full),
            pl.BlockSpec((1, W), full),
        ],
        out_specs=pl.BlockSpec((_BE, W), blk),
        out_shape=jax.ShapeDtypeStruct((EPAD, W), jnp.float32),
    )(xi, xj, W1cp, W1bp, b1, W2p, b2p)


def _hsum_body(p_ref, h_ref):
    h_ref[...] = p_ref[:N, :] + p_ref[N:, :]


def _hsum(parts):
    return pl.pallas_call(
        _hsum_body,
        out_shape=jax.ShapeDtypeStruct((N, W), jnp.float32),
    )(parts)


def _uv_body(p_ref, we2_ref, bias_ref, uv_ref):
    h2 = p_ref[:N, :] + p_ref[N:, :]
    uv = jnp.dot(h2.astype(jnp.bfloat16), we2_ref[...],
                 preferred_element_type=jnp.float32)
    uv_ref[...] = uv + bias_ref[...]


def _uv(parts, We2p, bias2):
    return pl.pallas_call(
        _uv_body,
        out_shape=jax.ShapeDtypeStruct((N, 2), jnp.float32),
    )(parts, We2p, bias2)


@functools.partial(
    pl.kernel,
    mesh=_mesh,
    out_type=(
        jax.ShapeDtypeStruct((EPAD, W), jnp.float32),
        jax.ShapeDtypeStruct((EPAD, W), jnp.float32),
    ),
    scratch_types=[
        pltpu.VMEM((EPWG,), jnp.int32),
        pltpu.VMEM((EPWG,), jnp.int32),
        pltpu.VMEM((2, BG, W), jnp.float32),
        pltpu.VMEM((2, BG, W), jnp.float32),
        pltpu.SemaphoreType.DMA,
        pltpu.SemaphoreType.DMA,
    ],
)
def _gather_k(h_hbm, dsti_hbm, srci_hbm, xi_hbm, xj_hbm,
              dstv, srcv, xib, xjb, gsem, ssem):
    wid = _wid()
    e0 = wid * EPWG
    pltpu.sync_copy(dsti_hbm.at[pl.ds(e0, EPWG)], dstv)
    pltpu.sync_copy(srci_hbm.at[pl.ds(e0, EPWG)], srcv)

    def fire(g, s):
        ix = pl.ds(g * BG, BG)
        pltpu.async_copy(h_hbm.at[dstv.at[ix]], xib.at[s], gsem)
        pltpu.async_copy(h_hbm.at[srcv.at[ix]], xjb.at[s], gsem)

    def drain_g(g, s):
        ix = pl.ds(g * BG, BG)
        pltpu.make_async_copy(h_hbm.at[dstv.at[ix]], xib.at[s], gsem).wait()
        pltpu.make_async_copy(h_hbm.at[srcv.at[ix]], xjb.at[s], gsem).wait()

    def fire_store(g, s):
        pltpu.async_copy(xib.at[s], xi_hbm.at[pl.ds(e0 + g * BG, BG)], ssem)
        pltpu.async_copy(xjb.at[s], xj_hbm.at[pl.ds(e0 + g * BG, BG)], ssem)

    def drain_store(s):
        pltpu.make_async_copy(xib.at[s], xi_hbm.at[pl.ds(e0, BG)], ssem).wait()
        pltpu.make_async_copy(xjb.at[s], xj_hbm.at[pl.ds(e0, BG)], ssem).wait()

    fire(0, 0)

    def body(g, carry):
        s = lax.rem(g, 2)

        @pl.when(g > 0)
        def _():
            drain_store(1 - s)

        drain_g(g, s)

        @pl.when(g < NGG - 1)
        def _():
            fire(g + 1, 1 - s)

        fire_store(g, s)
        return carry

    lax.fori_loop(0, NGG, body, 0)
    drain_store(lax.rem(NGG - 1, 2))


@functools.partial(
    pl.kernel,
    mesh=_mesh,
    out_type=jax.ShapeDtypeStruct((NC * N, W), jnp.float32),
    scratch_types=[
        pltpu.VMEM((NB, B), jnp.int32),
        pltpu.VMEM((GR, W), jnp.float32),
        pltpu.VMEM_SHARED((N, W), jnp.float32),
        pltpu.SemaphoreType.DMA,
    ],
)
def _scatter_k(m2_hbm, dsti_hbm, zero_hbm, out_hbm, dstv, mbuf, acc, sem):
    cid = lax.axis_index("c")
    sid = lax.axis_index("s")
    wid = sid * NC + cid
    base = sid * NPA
    pltpu.sync_copy(zero_hbm.at[pl.ds(base, NPA)], acc.at[pl.ds(base, NPA)])

    @pl.when(sid == NS - 1)
    def _init_tail():
        pltpu.sync_copy(zero_hbm.at[pl.ds(NS * NPA, NTAIL)],
                        acc.at[pl.ds(NS * NPA, NTAIL)])

    pltpu.sync_copy(dsti_hbm.at[wid], dstv)
    plsc.subcore_barrier()
    e0 = wid * EPW

    def body(g, carry):
        pltpu.async_copy(m2_hbm.at[pl.ds(e0 + g * GR, GR)], mbuf, sem).wait()
        for bi in range(GPB):
            pltpu.sync_copy(mbuf.at[pl.ds(bi * B, B)],
                            acc.at[dstv.at[g * GPB + bi]], add=True)
        return carry

    lax.fori_loop(0, NG, body, 0)
    plsc.subcore_barrier()
    pltpu.sync_copy(acc.at[pl.ds(base, NPA)],
                    out_hbm.at[pl.ds(cid * N + base, NPA)])

    @pl.when(sid == NS - 1)
    def _dump_tail():
        pltpu.sync_copy(acc.at[pl.ds(NS * NPA, NTAIL)],
                        out_hbm.at[pl.ds(cid * N + NS * NPA, NTAIL)])


@functools.partial(
    pl.kernel,
    mesh=_mesh,
    out_type=jax.ShapeDtypeStruct((NW, EPL, L), jnp.float32),
    scratch_types=[
        pltpu.VMEM((N, 2), jnp.float32),
        pltpu.VMEM((EPL, L), jnp.int32),
        pltpu.VMEM((EPL, L), jnp.int32),
        pltpu.VMEM((EPL, L), jnp.float32),
    ],
    compiler_params=pltpu.CompilerParams(use_tc_tiling_on_sc=False,
                                         needs_layout_passes=False),
)
def _final_k(uv_hbm, srci_hbm, dsti_hbm, out_hbm, uvv, srcv, dstv, obuf):
    wid = _wid()
    pltpu.sync_copy(uv_hbm, uvv)
    pltpu.sync_copy(srci_hbm.at[wid], srcv)
    pltpu.sync_copy(dsti_hbm.at[wid], dstv)
    col0 = jnp.zeros((L,), jnp.int32)
    col1 = jnp.ones((L,), jnp.int32)

    def body(j, carry):
        u = plsc.load_gather(uvv, [srcv[j], col0])
        v = plsc.load_gather(uvv, [dstv[j], col1])
        z = u + v
        obuf[j] = 1.0 / (1.0 + jnp.exp(-z))
        return carry

    lax.fori_loop(0, EPL, body, 0)
    pltpu.sync_copy(obuf, out_hbm.at[wid])


def kernel(x, edge_index, W_in, b_in, gamma, beta, W1, b1, W2, b2, We, be):
    f32 = jnp.float32
    src = edge_index[0]
    dst = edge_index[1]
    bf16 = jnp.bfloat16
    z64 = jnp.zeros((HID, W), f32)
    W1ap = jnp.concatenate([W1[:HID], z64], axis=0).astype(bf16)
    W1bp = jnp.concatenate([W1[HID:], z64], axis=0).astype(bf16)
    W2p = jnp.concatenate(
        [W2, jnp.zeros((2 * HID, HID), f32)], axis=1).astype(bf16)
    b2p = jnp.concatenate([b2, jnp.zeros((HID,), f32)])[None, :]
    We2p = jnp.concatenate(
        [jnp.concatenate([We[:HID], We[HID:]], axis=1),
         jnp.zeros((HID, 2), f32)], axis=0).astype(bf16)
    bias2 = jnp.concatenate([be, jnp.zeros((1,), f32)])[None, :]
    W_in_b = W_in.astype(bf16)

    pad = jnp.zeros((EPAD - E,), jnp.int32)
    srcp = jnp.concatenate([src, pad])
    dstp = jnp.concatenate([dst, pad])
    src4 = src.reshape(NW, NB, B)
    dst4 = dst.reshape(NW, NB, B)
    src5 = src.reshape(NW, EPL, L)
    dst5 = dst.reshape(NW, EPL, L)
    zeros_nw = jnp.zeros((N, W), f32)

    H = _prep(x, W_in_b, b_in[None, :], gamma[None, :], beta[None, :])
    uv = None
    for it in range(2):
        xi, xj = _gather_k(H, dstp, srcp)
        m2 = _mlp(xi, xj, W1ap, W1bp, b1[None, :], W2p, b2p)
        parts = _scatter_k(m2, dst4, zeros_nw)
        if it == 0:
            H = _hsum(parts)
        else:
            uv = _uv(parts, We2p, bias2)
    out = _final_k(uv, src5, dst5)
    return out.reshape(E)


# speedup vs baseline: 1.3857x; 1.3857x over previous
import functools

import jax
import jax.numpy as jnp
from jax import lax
from jax.experimental import pallas as pl
from jax.experimental.pallas import tpu as pltpu
from jax.experimental.pallas import tpu_sc as plsc

N = 10000
E = 320000
D = 128
HID = 64
W = 128

NC, NS, L = 2, 16, 16
NW = NC * NS
EPW = E // NW
B = 100
NB = EPW // B
GPB = 2
NG = NB // GPB
GR = GPB * B

BG = 128
EPAD = 327680
EPWG = EPAD // NW
NGG = EPWG // BG
EPL = EPW // L
NPT = N // NS
NPA = 624
NTAIL = N - NS * NPA

_mesh = plsc.VectorSubcoreMesh(core_axis_name="c", subcore_axis_name="s")


def _wid():
    return lax.axis_index("s") * NC + lax.axis_index("c")


def _prep_body(x_ref, win_ref, bin_ref, g_ref, b_ref, h_ref):
    h = jnp.dot(x_ref[...].astype(jnp.bfloat16), win_ref[...],
                preferred_element_type=jnp.float32)
    h = h + bin_ref[...]
    mean = jnp.mean(h, axis=0, keepdims=True)
    var = jnp.mean((h - mean) ** 2, axis=0, keepdims=True)
    hn = g_ref[...] * (h - mean) * lax.rsqrt(var + 1e-5) + b_ref[...]
    t = jnp.tanh(hn)
    h_ref[...] = jnp.concatenate([t, jnp.zeros_like(t)], axis=1)


def _prep(x, W_in, b_in, gamma, beta):
    return pl.pallas_call(
        _prep_body,
        out_shape=jax.ShapeDtypeStruct((N, W), jnp.float32),
    )(x, W_in, b_in, gamma, beta)


_BE = 2048


def _mlp_body(xi_ref, xj_ref, w1c_ref, w1b_ref, b1_ref, w2_ref, b2_ref, o_ref):
    xi = xi_ref[...]
    xj = xj_ref[...]
    m1 = jnp.dot(xi.astype(jnp.bfloat16), w1c_ref[...],
                 preferred_element_type=jnp.float32)
    m1 = m1 + jnp.dot((xj - xi).astype(jnp.bfloat16), w1b_ref[...],
                      preferred_element_type=jnp.float32)
    m1 = jnp.maximum(m1 + b1_ref[...], 0.0)
    m2 = jnp.dot(m1.astype(jnp.bfloat16), w2_ref[...],
                 preferred_element_type=jnp.float32)
    o_ref[...] = jnp.maximum(m2 + b2_ref[...], 0.0)


def _mlp(xi, xj, W1cp, W1bp, b1, W2p, b2p):
    nblk = EPAD // _BE
    blk = lambda i: (i, 0)
    full = lambda i: (0, 0)
    return pl.pallas_call(
        _mlp_body,
        grid=(nblk,),
        in_specs=[
            pl.BlockSpec((_BE, W), blk),
            pl.BlockSpec((_BE, W), blk),
            pl.BlockSpec((W, W), full),
            pl.BlockSpec((W, W), full),
            pl.BlockSpec((1, W), full),
            pl.BlockSpec((W, W), full),
            pl.BlockSpec((1, W), full),
        ],
        out_specs=pl.BlockSpec((_BE, W), blk),
        out_shape=jax.ShapeDtypeStruct((EPAD, W), jnp.float32),
    )(xi, xj, W1cp, W1bp, b1, W2p, b2p)


def _hsum_body(p_ref, h_ref):
    h_ref[...] = p_ref[:N, :] + p_ref[N:, :]


def _hsum(parts):
    return pl.pallas_call(
        _hsum_body,
        out_shape=jax.ShapeDtypeStruct((N, W), jnp.float32),
    )(parts)


def _uv_body(p_ref, we2_ref, bias_ref, uv_ref):
    h2 = p_ref[:N, :] + p_ref[N:, :]
    uv = jnp.dot(h2.astype(jnp.bfloat16), we2_ref[...],
                 preferred_element_type=jnp.float32)
    uv_ref[...] = uv + bias_ref[...]


def _uv(parts, We2p, bias2):
    return pl.pallas_call(
        _uv_body,
        out_shape=jax.ShapeDtypeStruct((N, 2), jnp.float32),
    )(parts, We2p, bias2)


@functools.partial(
    pl.kernel,
    mesh=_mesh,
    out_type=(
        jax.ShapeDtypeStruct((EPAD, W), jnp.float32),
        jax.ShapeDtypeStruct((EPAD, W), jnp.float32),
    ),
    scratch_types=[
        pltpu.VMEM((EPWG,), jnp.int32),
        pltpu.VMEM((EPWG,), jnp.int32),
        pltpu.VMEM((2, BG, W), jnp.float32),
        pltpu.VMEM((2, BG, W), jnp.float32),
        pltpu.SemaphoreType.DMA,
        pltpu.SemaphoreType.DMA,
    ],
)
def _gather_k(h_hbm, dsti_hbm, srci_hbm, xi_hbm, xj_hbm,
              dstv, srcv, xib, xjb, gsem, ssem):
    wid = _wid()
    e0 = wid * EPWG
    pltpu.sync_copy(dsti_hbm.at[pl.ds(e0, EPWG)], dstv)
    pltpu.sync_copy(srci_hbm.at[pl.ds(e0, EPWG)], srcv)

    def fire(g, s):
        ix = pl.ds(g * BG, BG)
        pltpu.async_copy(h_hbm.at[dstv.at[ix]], xib.at[s], gsem)
        pltpu.async_copy(h_hbm.at[srcv.at[ix]], xjb.at[s], gsem)

    def drain_g(g, s):
        ix = pl.ds(g * BG, BG)
        pltpu.make_async_copy(h_hbm.at[dstv.at[ix]], xib.at[s], gsem).wait()
        pltpu.make_async_copy(h_hbm.at[srcv.at[ix]], xjb.at[s], gsem).wait()

    def fire_store(g, s):
        pltpu.async_copy(xib.at[s], xi_hbm.at[pl.ds(e0 + g * BG, BG)], ssem)
        pltpu.async_copy(xjb.at[s], xj_hbm.at[pl.ds(e0 + g * BG, BG)], ssem)

    def drain_store(s):
        pltpu.make_async_copy(xib.at[s], xi_hbm.at[pl.ds(e0, BG)], ssem).wait()
        pltpu.make_async_copy(xjb.at[s], xj_hbm.at[pl.ds(e0, BG)], ssem).wait()

    fire(0, 0)

    def body(g, carry):
        s = lax.rem(g, 2)

        @pl.when(g > 0)
        def _():
            drain_store(1 - s)

        drain_g(g, s)

        @pl.when(g < NGG - 1)
        def _():
            fire(g + 1, 1 - s)

        fire_store(g, s)
        return carry

    lax.fori_loop(0, NGG, body, 0)
    drain_store(lax.rem(NGG - 1, 2))


@functools.partial(
    pl.kernel,
    mesh=_mesh,
    out_type=jax.ShapeDtypeStruct((NC * N, W), jnp.float32),
    scratch_types=[
        pltpu.VMEM((NB, B), jnp.int32),
        pltpu.VMEM((GR, W), jnp.float32),
        pltpu.VMEM_SHARED((N, W), jnp.float32),
        pltpu.SemaphoreType.DMA,
    ],
)
def _scatter_k(m2_hbm, dsti_hbm, zero_hbm, out_hbm, dstv, mbuf, acc, sem):
    cid = lax.axis_index("c")
    sid = lax.axis_index("s")
    wid = sid * NC + cid
    base = sid * NPA
    pltpu.sync_copy(zero_hbm.at[pl.ds(base, NPA)], acc.at[pl.ds(base, NPA)])

    @pl.when(sid == NS - 1)
    def _init_tail():
        pltpu.sync_copy(zero_hbm.at[pl.ds(NS * NPA, NTAIL)],
                        acc.at[pl.ds(NS * NPA, NTAIL)])

    pltpu.sync_copy(dsti_hbm.at[wid], dstv)
    plsc.subcore_barrier()
    e0 = wid * EPW

    def body(g, carry):
        pltpu.async_copy(m2_hbm.at[pl.ds(e0 + g * GR, GR)], mbuf, sem).wait()
        for bi in range(GPB):
            pltpu.sync_copy(mbuf.at[pl.ds(bi * B, B)],
                            acc.at[dstv.at[g * GPB + bi]], add=True)
        return carry

    lax.fori_loop(0, NG, body, 0)
    plsc.subcore_barrier()
    pltpu.sync_copy(acc.at[pl.ds(base, NPA)],
                    out_hbm.at[pl.ds(cid * N + base, NPA)])

    @pl.when(sid == NS - 1)
    def _dump_tail():
        pltpu.sync_copy(acc.at[pl.ds(NS * NPA, NTAIL)],
                        out_hbm.at[pl.ds(cid * N + NS * NPA, NTAIL)])


@functools.partial(
    pl.kernel,
    mesh=_mesh,
    out_type=jax.ShapeDtypeStruct((NW, EPL, L), jnp.float32),
    scratch_types=[
        pltpu.VMEM((N, 2), jnp.float32),
        pltpu.VMEM((EPL, L), jnp.int32),
        pltpu.VMEM((EPL, L), jnp.int32),
        pltpu.VMEM((EPL, L), jnp.float32),
    ],
    compiler_params=pltpu.CompilerParams(use_tc_tiling_on_sc=False,
                                         needs_layout_passes=False),
)
def _final_k(uv_hbm, srci_hbm, dsti_hbm, out_hbm, uvv, srcv, dstv, obuf):
    wid = _wid()
    pltpu.sync_copy(uv_hbm, uvv)
    pltpu.sync_copy(srci_hbm.at[wid], srcv)
    pltpu.sync_copy(dsti_hbm.at[wid], dstv)
    col0 = jnp.zeros((L,), jnp.int32)
    col1 = jnp.ones((L,), jnp.int32)

    def body(j, carry):
        u = plsc.load_gather(uvv, [srcv[j], col0])
        v = plsc.load_gather(uvv, [dstv[j], col1])
        z = u + v
        obuf[j] = 1.0 / (1.0 + jnp.exp(-z))
        return carry

    lax.fori_loop(0, EPL, body, 0)
    pltpu.sync_copy(obuf, out_hbm.at[wid])


def kernel(x, edge_index, W_in, b_in, gamma, beta, W1, b1, W2, b2, We, be):
    f32 = jnp.float32
    src = edge_index[0]
    dst = edge_index[1]
    bf16 = jnp.bfloat16
    z64 = jnp.zeros((HID, W), f32)
    W1ap = jnp.concatenate([W1[:HID], z64], axis=0).astype(bf16)
    W1bp = jnp.concatenate([W1[HID:], z64], axis=0).astype(bf16)
    W2p = jnp.concatenate(
        [W2, jnp.zeros((2 * HID, HID), f32)], axis=1).astype(bf16)
    b2p = jnp.concatenate([b2, jnp.zeros((HID,), f32)])[None, :]
    We2p = jnp.concatenate(
        [jnp.concatenate([We[:HID], We[HID:]], axis=1),
         jnp.zeros((HID, 2), f32)], axis=0).astype(bf16)
    bias2 = jnp.concatenate([be, jnp.zeros((1,), f32)])[None, :]
    W_in_b = W_in.astype(bf16)

    pad = (jnp.arange(EPAD - E, dtype=jnp.int32)) % N
    srcp = jnp.concatenate([src, pad])
    dstp = jnp.concatenate([dst, pad])
    src4 = src.reshape(NW, NB, B)
    dst4 = dst.reshape(NW, NB, B)
    src5 = src.reshape(NW, EPL, L)
    dst5 = dst.reshape(NW, EPL, L)
    zeros_nw = jnp.zeros((N, W), f32)

    H = _prep(x, W_in_b, b_in[None, :], gamma[None, :], beta[None, :])
    uv = None
    for it in range(2):
        xi, xj = _gather_k(H, dstp, srcp)
        m2 = _mlp(xi, xj, W1ap, W1bp, b1[None, :], W2p, b2p)
        parts = _scatter_k(m2, dst4, zeros_nw)
        if it == 0:
            H = _hsum(parts)
        else:
            uv = _uv(parts, We2p, bias2)
    out = _final_k(uv, src5, dst5)
    return out.reshape(E)


# speedup vs baseline: 1.5925x; 1.1492x over previous
import functools

import jax
import jax.numpy as jnp
from jax import lax
from jax.experimental import pallas as pl
from jax.experimental.pallas import tpu as pltpu
from jax.experimental.pallas import tpu_sc as plsc

N = 10000
E = 320000
D = 128
HID = 64
W = 128

NC, NS, L = 2, 16, 16
NW = NC * NS
EPW = E // NW

SEH = E // 2
EPW_S = SEH // NW
B = 100
NB = EPW_S // B
GPB = 2
NG = NB // GPB
GR = GPB * B

BG = 128
EH = 163840
EPWG = EH // NW
NGG = EPWG // BG
EPL = EPW // L
NPT = N // NS
NPA = 624
NTAIL = N - NS * NPA

_mesh = plsc.VectorSubcoreMesh(core_axis_name="c", subcore_axis_name="s")


def _wid():
    return lax.axis_index("s") * NC + lax.axis_index("c")


def _prep_body(x_ref, win_ref, bin_ref, g_ref, b_ref, h_ref):
    h = jnp.dot(x_ref[...].astype(jnp.bfloat16), win_ref[...],
                preferred_element_type=jnp.float32)
    h = h + bin_ref[...]
    mean = jnp.mean(h, axis=0, keepdims=True)
    var = jnp.mean((h - mean) ** 2, axis=0, keepdims=True)
    hn = g_ref[...] * (h - mean) * lax.rsqrt(var + 1e-5) + b_ref[...]
    t = jnp.tanh(hn)
    h_ref[...] = jnp.concatenate([t, jnp.zeros_like(t)], axis=1)


def _prep(x, W_in, b_in, gamma, beta):
    return pl.pallas_call(
        _prep_body,
        out_shape=jax.ShapeDtypeStruct((N, W), jnp.float32),
    )(x, W_in, b_in, gamma, beta)


_BE = 2048


def _mlp_body(xi_ref, xj_ref, w1c_ref, w1b_ref, b1_ref, w2_ref, b2_ref, o_ref):
    xi = xi_ref[...]
    xj = xj_ref[...]
    m1 = jnp.dot(xi.astype(jnp.bfloat16), w1c_ref[...],
                 preferred_element_type=jnp.float32)
    m1 = m1 + jnp.dot((xj - xi).astype(jnp.bfloat16), w1b_ref[...],
                      preferred_element_type=jnp.float32)
    m1 = jnp.maximum(m1 + b1_ref[...], 0.0)
    m2 = jnp.dot(m1.astype(jnp.bfloat16), w2_ref[...],
                 preferred_element_type=jnp.float32)
    o_ref[...] = jnp.maximum(m2 + b2_ref[...], 0.0)


def _mlp(xi, xj, W1cp, W1bp, b1, W2p, b2p):
    nblk = EH // _BE
    blk = lambda i: (i, 0)
    full = lambda i: (0, 0)
    return pl.pallas_call(
        _mlp_body,
        grid=(nblk,),
        in_specs=[
            pl.BlockSpec((_BE, W), blk),
            pl.BlockSpec((_BE, W), blk),
            pl.BlockSpec((W, W), full),
            pl.BlockSpec((W, W), full),
            pl.BlockSpec((1, W), full),
            pl.BlockSpec((W, W), full),
            pl.BlockSpec((1, W), full),
        ],
        out_specs=pl.BlockSpec((_BE, W), blk),
        out_shape=jax.ShapeDtypeStruct((EH, W), jnp.float32),
    )(xi, xj, W1cp, W1bp, b1, W2p, b2p)


def _hsum_body(p0_ref, p1_ref, h_ref):
    h_ref[...] = ((p0_ref[:N, :] + p0_ref[N:, :])
                  + (p1_ref[:N, :] + p1_ref[N:, :]))


def _hsum(p0, p1):
    return pl.pallas_call(
        _hsum_body,
        out_shape=jax.ShapeDtypeStruct((N, W), jnp.float32),
    )(p0, p1)


def _uv_body(p0_ref, p1_ref, we2_ref, bias_ref, uv_ref):
    h2 = ((p0_ref[:N, :] + p0_ref[N:, :])
          + (p1_ref[:N, :] + p1_ref[N:, :]))
    uv = jnp.dot(h2.astype(jnp.bfloat16), we2_ref[...],
                 preferred_element_type=jnp.float32)
    uv_ref[...] = uv + bias_ref[...]


def _uv(p0, p1, We2p, bias2):
    return pl.pallas_call(
        _uv_body,
        out_shape=jax.ShapeDtypeStruct((N, 2), jnp.float32),
    )(p0, p1, We2p, bias2)


@functools.partial(
    pl.kernel,
    mesh=_mesh,
    out_type=(
        jax.ShapeDtypeStruct((EH, W), jnp.float32),
        jax.ShapeDtypeStruct((EH, W), jnp.float32),
    ),
    scratch_types=[
        pltpu.VMEM((EPWG,), jnp.int32),
        pltpu.VMEM((EPWG,), jnp.int32),
        pltpu.VMEM((2, BG, W), jnp.float32),
        pltpu.VMEM((2, BG, W), jnp.float32),
        pltpu.SemaphoreType.DMA,
        pltpu.SemaphoreType.DMA,
    ],
)
def _gather_k(h_hbm, dsti_hbm, srci_hbm, xi_hbm, xj_hbm,
              dstv, srcv, xib, xjb, gsem, ssem):
    wid = _wid()
    e0 = wid * EPWG
    pltpu.sync_copy(dsti_hbm.at[pl.ds(e0, EPWG)], dstv)
    pltpu.sync_copy(srci_hbm.at[pl.ds(e0, EPWG)], srcv)

    def fire(g, s):
        ix = pl.ds(g * BG, BG)
        pltpu.async_copy(h_hbm.at[dstv.at[ix]], xib.at[s], gsem)
        pltpu.async_copy(h_hbm.at[srcv.at[ix]], xjb.at[s], gsem)

    def drain_g(g, s):
        ix = pl.ds(g * BG, BG)
        pltpu.make_async_copy(h_hbm.at[dstv.at[ix]], xib.at[s], gsem).wait()
        pltpu.make_async_copy(h_hbm.at[srcv.at[ix]], xjb.at[s], gsem).wait()

    def fire_store(g, s):
        pltpu.async_copy(xib.at[s], xi_hbm.at[pl.ds(e0 + g * BG, BG)], ssem)
        pltpu.async_copy(xjb.at[s], xj_hbm.at[pl.ds(e0 + g * BG, BG)], ssem)

    def drain_store(s):
        pltpu.make_async_copy(xib.at[s], xi_hbm.at[pl.ds(e0, BG)], ssem).wait()
        pltpu.make_async_copy(xjb.at[s], xj_hbm.at[pl.ds(e0, BG)], ssem).wait()

    fire(0, 0)

    def body(g, carry):
        s = lax.rem(g, 2)

        @pl.when(g > 0)
        def _():
            drain_store(1 - s)

        drain_g(g, s)

        @pl.when(g < NGG - 1)
        def _():
            fire(g + 1, 1 - s)

        fire_store(g, s)
        return carry

    lax.fori_loop(0, NGG, body, 0)
    drain_store(lax.rem(NGG - 1, 2))


@functools.partial(
    pl.kernel,
    mesh=_mesh,
    out_type=jax.ShapeDtypeStruct((NC * N, W), jnp.float32),
    scratch_types=[
        pltpu.VMEM((NB, B), jnp.int32),
        pltpu.VMEM((GR, W), jnp.float32),
        pltpu.VMEM_SHARED((N, W), jnp.float32),
        pltpu.SemaphoreType.DMA,
    ],
)
def _scatter_k(m2_hbm, dsti_hbm, zero_hbm, out_hbm, dstv, mbuf, acc, sem):
    cid = lax.axis_index("c")
    sid = lax.axis_index("s")
    wid = sid * NC + cid
    base = sid * NPA
    pltpu.sync_copy(zero_hbm.at[pl.ds(base, NPA)], acc.at[pl.ds(base, NPA)])

    @pl.when(sid == NS - 1)
    def _init_tail():
        pltpu.sync_copy(zero_hbm.at[pl.ds(NS * NPA, NTAIL)],
                        acc.at[pl.ds(NS * NPA, NTAIL)])

    pltpu.sync_copy(dsti_hbm.at[wid], dstv)
    plsc.subcore_barrier()
    e0 = wid * EPW_S

    def body(g, carry):
        pltpu.async_copy(m2_hbm.at[pl.ds(e0 + g * GR, GR)], mbuf, sem).wait()
        for bi in range(GPB):
            pltpu.sync_copy(mbuf.at[pl.ds(bi * B, B)],
                            acc.at[dstv.at[g * GPB + bi]], add=True)
        return carry

    lax.fori_loop(0, NG, body, 0)
    plsc.subcore_barrier()
    pltpu.sync_copy(acc.at[pl.ds(base, NPA)],
                    out_hbm.at[pl.ds(cid * N + base, NPA)])

    @pl.when(sid == NS - 1)
    def _dump_tail():
        pltpu.sync_copy(acc.at[pl.ds(NS * NPA, NTAIL)],
                        out_hbm.at[pl.ds(cid * N + NS * NPA, NTAIL)])


@functools.partial(
    pl.kernel,
    mesh=_mesh,
    out_type=jax.ShapeDtypeStruct((NW, EPL, L), jnp.float32),
    scratch_types=[
        pltpu.VMEM((N, 2), jnp.float32),
        pltpu.VMEM((EPL, L), jnp.int32),
        pltpu.VMEM((EPL, L), jnp.int32),
        pltpu.VMEM((EPL, L), jnp.float32),
    ],
    compiler_params=pltpu.CompilerParams(use_tc_tiling_on_sc=False,
                                         needs_layout_passes=False),
)
def _final_k(uv_hbm, srci_hbm, dsti_hbm, out_hbm, uvv, srcv, dstv, obuf):
    wid = _wid()
    pltpu.sync_copy(uv_hbm, uvv)
    pltpu.sync_copy(srci_hbm.at[wid], srcv)
    pltpu.sync_copy(dsti_hbm.at[wid], dstv)
    col0 = jnp.zeros((L,), jnp.int32)
    col1 = jnp.ones((L,), jnp.int32)

    def body(j, carry):
        u = plsc.load_gather(uvv, [srcv[j], col0])
        v = plsc.load_gather(uvv, [dstv[j], col1])
        z = u + v
        obuf[j] = 1.0 / (1.0 + jnp.exp(-z))
        return carry

    lax.fori_loop(0, EPL, body, 0)
    pltpu.sync_copy(obuf, out_hbm.at[wid])


def kernel(x, edge_index, W_in, b_in, gamma, beta, W1, b1, W2, b2, We, be):
    f32 = jnp.float32
    src = edge_index[0]
    dst = edge_index[1]
    bf16 = jnp.bfloat16
    z64 = jnp.zeros((HID, W), f32)
    W1ap = jnp.concatenate([W1[:HID], z64], axis=0).astype(bf16)
    W1bp = jnp.concatenate([W1[HID:], z64], axis=0).astype(bf16)
    W2p = jnp.concatenate(
        [W2, jnp.zeros((2 * HID, HID), f32)], axis=1).astype(bf16)
    b2p = jnp.concatenate([b2, jnp.zeros((HID,), f32)])[None, :]
    We2p = jnp.concatenate(
        [jnp.concatenate([We[:HID], We[HID:]], axis=1),
         jnp.zeros((HID, 2), f32)], axis=0).astype(bf16)
    bias2 = jnp.concatenate([be, jnp.zeros((1,), f32)])[None, :]
    W_in_b = W_in.astype(bf16)

    pad = (jnp.arange(EH - SEH, dtype=jnp.int32)) % N
    srch = [jnp.concatenate([src[h * SEH:(h + 1) * SEH], pad]) for h in (0, 1)]
    dsth = [jnp.concatenate([dst[h * SEH:(h + 1) * SEH], pad]) for h in (0, 1)]
    src4 = src.reshape(2, NW, NB, B)
    dst4 = dst.reshape(2, NW, NB, B)
    src5 = src.reshape(NW, EPL, L)
    dst5 = dst.reshape(NW, EPL, L)
    zeros_nw = jnp.zeros((N, W), f32)
    b1r = b1[None, :]

    H = _prep(x, W_in_b, b_in[None, :], gamma[None, :], beta[None, :])
    uv = None
    for it in range(2):
        xi0, xj0 = _gather_k(H, dsth[0], srch[0])
        xi1, xj1 = _gather_k(H, dsth[1], srch[1])
        m0 = _mlp(xi0, xj0, W1ap, W1bp, b1r, W2p, b2p)
        m1 = _mlp(xi1, xj1, W1ap, W1bp, b1r, W2p, b2p)
        p0 = _scatter_k(m0, dst4[0], zeros_nw)
        p1 = _scatter_k(m1, dst4[1], zeros_nw)
        if it == 0:
            H = _hsum(p0, p1)
        else:
            uv = _uv(p0, p1, We2p, bias2)
    out = _final_k(uv, src5, dst5)
    return out.reshape(E)
